# Initial kernel scaffold; baseline (speedup 1.0000x reference)
import jax
import jax.numpy as jnp
from jax.experimental import pallas as pl


def kernel(x, edge_index, edge_attr, Wq, Wk, Wv, Wo, Wek, Wev, Wg1, Wg2):
    raise NotImplementedError("write your pallas kernel here")


# speedup vs baseline: 7.5154x; 7.5154x over previous
import functools
import math

import jax
import jax.numpy as jnp
from jax import lax
from jax.experimental import pallas as pl
from jax.experimental.pallas import tpu as pltpu
from jax.experimental.pallas import tpu_sc as plsc

N = 10000
E = 320000
D = 128
H = 16
DH = 8
D_EDGE = 16
D_GATE = 64

NC = 2
NS = 16
NW = NC * NS
G = 32
NGROUPS = E // G
TRIPS = -(-NGROUPS // NW)
SUB = G // 16
ROWS_PT = 624
TAIL = N - ROWS_PT * NS

_INV_SQRT_DH = 1.0 / math.sqrt(DH)


def _qkv_body(x_ref, wq_ref, wk_ref, wv_ref, q_ref, k_ref, v_ref):
    x = x_ref[...]
    dn = (((1,), (1,)), ((), ()))
    q_ref[...] = lax.dot_general(x, wq_ref[...], dn,
                                 preferred_element_type=jnp.float32)
    k_ref[...] = lax.dot_general(x, wk_ref[...], dn,
                                 preferred_element_type=jnp.float32)
    v_ref[...] = lax.dot_general(x, wv_ref[...], dn,
                                 preferred_element_type=jnp.float32)


def _qkv(x, Wq, Wk, Wv):
    BN = 1000
    grid = (N // BN,)
    wspec = pl.BlockSpec((D, D), lambda i: (0, 0))
    return pl.pallas_call(
        _qkv_body,
        grid=grid,
        in_specs=[pl.BlockSpec((BN, D), lambda i: (i, 0)), wspec, wspec, wspec],
        out_specs=[pl.BlockSpec((BN, D), lambda i: (i, 0))] * 3,
        out_shape=[jax.ShapeDtypeStruct((N, D), jnp.float32)] * 3,
    )(x, Wq, Wk, Wv)


def _edge_body(ea_ref, wek_ref, wev_ref, wg1_ref, wg2_ref,
               ek_ref, ev_ref, gate_ref):
    ea = ea_ref[...]
    dn = (((1,), (1,)), ((), ()))
    ek_ref[...] = lax.dot_general(ea, wek_ref[...], dn,
                                  preferred_element_type=jnp.float32)
    ev_ref[...] = lax.dot_general(ea, wev_ref[...], dn,
                                  preferred_element_type=jnp.float32)
    g1 = lax.dot_general(ea, wg1_ref[...], dn,
                         preferred_element_type=jnp.float32)
    g1 = g1 * jax.nn.sigmoid(g1)
    gate_ref[...] = lax.dot_general(g1, wg2_ref[...], dn,
                                    preferred_element_type=jnp.float32)


def _edge_dense(edge_attr, Wek, Wev, Wg1, Wg2):
    BE = 4000
    grid = (E // BE,)
    return pl.pallas_call(
        _edge_body,
        grid=grid,
        in_specs=[
            pl.BlockSpec((BE, D_EDGE), lambda i: (i, 0)),
            pl.BlockSpec((D, D_EDGE), lambda i: (0, 0)),
            pl.BlockSpec((D, D_EDGE), lambda i: (0, 0)),
            pl.BlockSpec((D_GATE, D_EDGE), lambda i: (0, 0)),
            pl.BlockSpec((H, D_GATE), lambda i: (0, 0)),
        ],
        out_specs=[
            pl.BlockSpec((BE, D), lambda i: (i, 0)),
            pl.BlockSpec((BE, D), lambda i: (i, 0)),
            pl.BlockSpec((BE, H), lambda i: (i, 0)),
        ],
        out_shape=[
            jax.ShapeDtypeStruct((E, D), jnp.float32),
            jax.ShapeDtypeStruct((E, D), jnp.float32),
            jax.ShapeDtypeStruct((E, H), jnp.float32),
        ],
    )(edge_attr, Wek, Wev, Wg1, Wg2)


N8 = N // 8
N8P = 1280


def _sc_body(q_hbm, k_hbm, v_hbm, ek_hbm, ev_hbm, gate_hbm,
             src_hbm, dst_hbm, z128_hbm,
             outS_hbm, outZ_hbm,
             srcv, dstv, zdiv, qe, ke, eke, ve, eve, gbuf, zstage,
             s_acc, z_acc,
             sem_q, sem_k, sem_v):
    c = lax.axis_index("c")
    s = lax.axis_index("s")
    wid = c * NS + s

    iota16 = lax.iota(jnp.int32, 16)
    rb = s * ROWS_PT
    CH = 16
    NCH = ROWS_PT // CH

    pltpu.sync_copy(z128_hbm.at[pl.ds(0, G)], zstage)
    pltpu.sync_copy(z128_hbm.at[pl.ds(0, CH)], qe.at[pl.ds(0, CH)])

    @pl.loop(0, NCH)
    def _init(i):
        idxv = iota16 + (rb + i * CH)
        pltpu.sync_copy(qe.at[pl.ds(0, CH)], s_acc.at[idxv])

    @pl.when(s == NS - 1)
    def _tail_init():
        idxv = iota16 + (ROWS_PT * NS)
        pltpu.sync_copy(qe.at[pl.ds(0, CH)], s_acc.at[idxv])

    zb = s * (N8P // NS)
    for off in (0, 16, 32, 48, 64):
        pltpu.sync_copy(qe.at[pl.ds(0, CH)], z_acc.at[iota16 + (zb + off)])

    plsc.subcore_barrier()

    @pl.loop(0, TRIPS)
    def _grp(t):
        g = wid + NW * t

        @pl.when(g < NGROUPS)
        def _do_group():
            eb = g * G
            pltpu.sync_copy(src_hbm.at[pl.ds(eb, G)], srcv)
            pltpu.sync_copy(dst_hbm.at[pl.ds(eb, G)], dstv)
            cq = pltpu.async_copy(q_hbm.at[dstv], qe, sem_q)
            ck = pltpu.async_copy(k_hbm.at[srcv], ke, sem_k)
            cv = pltpu.async_copy(v_hbm.at[srcv], ve, sem_v)
            pltpu.sync_copy(ek_hbm.at[pl.ds(eb, G)], eke)
            pltpu.sync_copy(ev_hbm.at[pl.ds(eb, G)], eve)
            pltpu.sync_copy(gate_hbm.at[pl.ds(eb, G)], gbuf)
            cq.wait()
            ck.wait()
            cv.wait()

            for sub in range(SUB):
                rowv = iota16 + sub * 16
                dv = dstv[pl.ds(sub * 16, 16)]
                zdiv[pl.ds(sub * 16, 16)] = lax.shift_right_logical(dv, 3)
                dcol = (dv & 7) * 16

                @pl.loop(0, H)
                def _score(h, rowv=rowv, dcol=dcol):
                    acc = jnp.zeros((16,), jnp.float32)
                    for dd in range(DH):
                        col = jnp.full((16,), h * DH + dd, jnp.int32)
                        acc = acc + plsc.load_gather(qe, [rowv, col]) * (
                            plsc.load_gather(ke, [rowv, col])
                            + plsc.load_gather(eke, [rowv, col]))
                    hcol = jnp.full((16,), h, jnp.int32)
                    sv = acc * _INV_SQRT_DH + plsc.load_gather(gbuf, [rowv, hcol])
                    plsc.store_scatter(zstage, [rowv, dcol + h], jnp.exp(sv))

                @pl.loop(0, H)
                def _wv(h, rowv=rowv, dcol=dcol):
                    e_h = plsc.load_gather(zstage, [rowv, dcol + h])
                    for dd in range(DH):
                        col = jnp.full((16,), h * DH + dd, jnp.int32)
                        w = e_h * (plsc.load_gather(ve, [rowv, col])
                                   + plsc.load_gather(eve, [rowv, col]))
                        plsc.store_scatter(ve, [rowv, col], w)

            pltpu.sync_copy(zstage, z_acc.at[zdiv], add=True)
            pltpu.sync_copy(ve, s_acc.at[dstv], add=True)

            zero16 = jnp.zeros((16,), jnp.float32)
            for sub in range(SUB):
                rowv = iota16 + sub * 16
                dv = dstv[pl.ds(sub * 16, 16)]
                dcol = (dv & 7) * 16

                @pl.loop(0, H)
                def _clear(h, rowv=rowv, dcol=dcol):
                    plsc.store_scatter(zstage, [rowv, dcol + h], zero16)

    plsc.subcore_barrier()

    @pl.loop(0, NCH)
    def _dump(i):
        idxv = iota16 + (rb + i * CH)
        pltpu.sync_copy(s_acc.at[idxv], qe.at[pl.ds(0, CH)])
        pltpu.sync_copy(qe.at[pl.ds(0, CH)], outS_hbm.at[c, pl.ds(rb + i * CH, CH)])

    @pl.when(s == NS - 1)
    def _tail_dump():
        tb = ROWS_PT * NS
        idxv = iota16 + tb
        pltpu.sync_copy(s_acc.at[idxv], qe.at[pl.ds(0, TAIL)])
        pltpu.sync_copy(qe.at[pl.ds(0, TAIL)], outS_hbm.at[c, pl.ds(tb, TAIL)])

    for off in (0, 16, 32, 48, 64):
        idxv = iota16 + (zb + off)
        pltpu.sync_copy(z_acc.at[idxv], ke.at[pl.ds(0, CH)])
        pltpu.sync_copy(ke.at[pl.ds(0, CH)], outZ_hbm.at[c, pl.ds(zb + off, CH)])


def _sc_pass(q, k, v, ek, ev, gate, src, dst):
    z128 = jnp.zeros((N, D), jnp.float32)
    mesh = plsc.VectorSubcoreMesh(core_axis_name="c", subcore_axis_name="s",
                                  num_cores=NC, num_subcores=NS)
    f = pl.kernel(
        _sc_body,
        compiler_params=pltpu.CompilerParams(needs_layout_passes=False),
        out_type=[
            jax.ShapeDtypeStruct((NC, N, D), jnp.float32),
            jax.ShapeDtypeStruct((NC, N8P, D), jnp.float32),
        ],
        mesh=mesh,
        scratch_types=[
            pltpu.VMEM((G,), jnp.int32),
            pltpu.VMEM((G,), jnp.int32),
            pltpu.VMEM((G,), jnp.int32),
            pltpu.VMEM((G, D), jnp.float32),
            pltpu.VMEM((G, D), jnp.float32),
            pltpu.VMEM((G, D), jnp.float32),
            pltpu.VMEM((G, D), jnp.float32),
            pltpu.VMEM((G, D), jnp.float32),
            pltpu.VMEM((G, H), jnp.float32),
            pltpu.VMEM((G, D), jnp.float32),
            pltpu.VMEM_SHARED((N, D), jnp.float32),
            pltpu.VMEM_SHARED((N8P, D), jnp.float32),
            pltpu.SemaphoreType.DMA,
            pltpu.SemaphoreType.DMA,
            pltpu.SemaphoreType.DMA,
        ],
    )
    Sp, Zp = f(q, k, v, ek, ev, gate, src, dst, z128)
    return Sp, Zp[:, :N8].reshape(NC, N, H)


def _final_body(s0_ref, s1_ref, zz0_ref, zz1_ref, rc_ref, wo_ref, o_ref):
    ssum = s0_ref[0] + s1_ref[0]
    zsum = zz0_ref[0] + zz1_ref[0] + 1e-9
    recip = 1.0 / zsum
    dn = (((1,), (1,)), ((), ()))
    zrep = lax.dot_general(recip, rc_ref[...], dn,
                           preferred_element_type=jnp.float32)
    o_ref[...] = lax.dot_general(ssum * zrep, wo_ref[...], dn,
                                 preferred_element_type=jnp.float32)


def _final(Sp, Zp, Wo):
    BN = 1000
    grid = (N // BN,)
    rc = (jnp.arange(D)[:, None] // DH == jnp.arange(H)[None, :]).astype(jnp.float32)
    return pl.pallas_call(
        _final_body,
        grid=grid,
        in_specs=[
            pl.BlockSpec((1, BN, D), lambda i: (0, i, 0)),
            pl.BlockSpec((1, BN, D), lambda i: (1, i, 0)),
            pl.BlockSpec((1, BN, H), lambda i: (0, i, 0)),
            pl.BlockSpec((1, BN, H), lambda i: (1, i, 0)),
            pl.BlockSpec((D, H), lambda i: (0, 0)),
            pl.BlockSpec((D, D), lambda i: (0, 0)),
        ],
        out_specs=pl.BlockSpec((BN, D), lambda i: (i, 0)),
        out_shape=jax.ShapeDtypeStruct((N, D), jnp.float32),
    )(Sp, Sp, Zp, Zp, rc, Wo)


def kernel(x, edge_index, edge_attr, Wq, Wk, Wv, Wo, Wek, Wev, Wg1, Wg2):
    ei = edge_index.astype(jnp.int32)
    src = ei[0]
    dst = ei[1]
    q, k, v = _qkv(x, Wq, Wk, Wv)
    ek, ev, gate = _edge_dense(edge_attr, Wek, Wev, Wg1, Wg2)
    Sp, Zp = _sc_pass(q, k, v, ek, ev, gate, src, dst)
    return _final(Sp, Zp, Wo)
